# Initial kernel scaffold; baseline (speedup 1.0000x reference)
import functools

import numpy as np
import jax
import jax.numpy as jnp
from jax import lax
from jax.experimental import pallas as pl
from jax.experimental.pallas import tpu as pltpu
from jax.experimental.pallas import tpu_sc as plsc

_N = 10000
_NP = 10240
_IN_C = 256
_HID = 256
_OUT = 128
_K = 10
_BN_EPS = 1e-5

_NC, _NS = 2, 16
_CH = 128
_CPT = 81
_EP = _NC * _NS * _CPT * _CH
_ROWS_PT = _NP // _NS
_NBUF = 4

_MESH = plsc.VectorSubcoreMesh(
    core_axis_name="c", subcore_axis_name="s", num_cores=_NC, num_subcores=_NS)


def _fill_buf(buf, value):
    vec = jnp.full((16,), value, jnp.float32)

    def body(i, _):
        for l in range(_OUT // 16):
            buf[i, pl.ds(l * 16, 16)] = vec
        return 0

    lax.fori_loop(0, _CH, body, 0)


@functools.partial(
    pl.kernel,
    out_type=jax.ShapeDtypeStruct((_NC, _NP, _OUT), jnp.float32),
    mesh=_MESH,
    scratch_types=[
        pltpu.VMEM((_CPT, _CH), jnp.int32),
        pltpu.VMEM((2, _CH, _OUT), jnp.float32),
        pltpu.VMEM_SHARED((_NP, _OUT), jnp.float32),
        pltpu.SemaphoreType.DMA,
    ],
)
def _deg_kernel(dst_hbm, out_hbm, dst_v, zo, acc_sh, sem):
    c = lax.axis_index("c")
    s = lax.axis_index("s")
    pltpu.sync_copy(dst_hbm.at[c, s], dst_v)
    _fill_buf(zo.at[0], 0.0)
    _fill_buf(zo.at[1], 1.0)
    for r in range(_ROWS_PT // _CH):
        pltpu.sync_copy(zo.at[0], acc_sh.at[pl.ds(s * _ROWS_PT + r * _CH, _CH)])
    plsc.subcore_barrier()

    def start(j):
        pltpu.async_copy(zo.at[1], acc_sh.at[dst_v.at[j]], sem, add=True)

    def drain(j):
        pltpu.make_async_copy(zo.at[1], acc_sh.at[dst_v.at[j]], sem).wait()

    def group(gi, _):
        for q in range(8):
            start(gi * 8 + q)
        for q in range(8):
            drain(gi * 8 + q)
        return 0

    lax.fori_loop(0, _CPT // 8, group, 0)
    for j in range((_CPT // 8) * 8, _CPT):
        start(j)
        drain(j)
    plsc.subcore_barrier()
    pltpu.sync_copy(acc_sh.at[pl.ds(s * _ROWS_PT, _ROWS_PT)],
                    out_hbm.at[c, pl.ds(s * _ROWS_PT, _ROWS_PT)])


@functools.partial(
    pl.kernel,
    out_type=jax.ShapeDtypeStruct((_NC, _NP, _OUT), jnp.float32),
    mesh=_MESH,
    scratch_types=[
        pltpu.VMEM((_CPT, _CH), jnp.int32),
        pltpu.VMEM((_CPT, _CH), jnp.int32),
        pltpu.VMEM((_NBUF, _CH, _OUT), jnp.float32),
        pltpu.VMEM_SHARED((_NP, _OUT), jnp.float32),
        pltpu.SemaphoreType.DMA((_NBUF,)),
        pltpu.SemaphoreType.DMA((_NBUF,)),
    ],
)
def _edge_kernel(g_hbm, src_hbm, dst_hbm, out_hbm,
                 src_v, dst_v, rows, acc_sh, gs, ss):
    c = lax.axis_index("c")
    s = lax.axis_index("s")
    pltpu.sync_copy(src_hbm.at[c, s], src_v)
    pltpu.sync_copy(dst_hbm.at[c, s], dst_v)
    _fill_buf(rows.at[0], 0.0)
    for r in range(_ROWS_PT // _CH):
        pltpu.sync_copy(rows.at[0], acc_sh.at[pl.ds(s * _ROWS_PT + r * _CH, _CH)])
    plsc.subcore_barrier()

    def g_start(j, b):
        pltpu.async_copy(g_hbm.at[src_v.at[j]], rows.at[b], gs.at[b])

    def g_wait(j, b):
        pltpu.make_async_copy(g_hbm.at[src_v.at[j]], rows.at[b], gs.at[b]).wait()

    def s_start(j, b):
        pltpu.async_copy(rows.at[b], acc_sh.at[dst_v.at[j]], ss.at[b], add=True)

    def s_wait(j, b):
        pltpu.make_async_copy(rows.at[b], acc_sh.at[dst_v.at[j]], ss.at[b]).wait()

    g_start(0, 0)
    g_start(1, 1)
    g_wait(0, 0); s_start(0, 0); g_start(2, 2)
    g_wait(1, 1); s_start(1, 1); g_start(3, 3)
    g_wait(2, 2); s_start(2, 2); s_wait(0, 0); g_start(4, 0)
    g_wait(3, 3); s_start(3, 3); s_wait(1, 1); g_start(5, 1)

    def steady(m, _):
        for q in range(4):
            j = 4 * m + 4 + q
            g_wait(j, q)
            s_start(j, q)
            s_wait(j - 2, (q + 2) % 4)

            @pl.when(j <= _CPT - 3)
            def _():
                g_start(j + 2, (q + 2) % 4)
        return 0

    lax.fori_loop(0, (_CPT - 5) // 4, steady, 0)
    g_wait(80, 0); s_start(80, 0); s_wait(78, 2)
    s_wait(79, 3)
    s_wait(80, 0)
    plsc.subcore_barrier()
    pltpu.sync_copy(acc_sh.at[pl.ds(s * _ROWS_PT, _ROWS_PT)],
                    out_hbm.at[c, pl.ds(s * _ROWS_PT, _ROWS_PT)])


_BM = 1024


def _dinv_body(deg_ref, out_ref):
    i = pl.program_id(0)
    d = deg_ref[0] + deg_ref[1]
    row = lax.broadcasted_iota(jnp.int32, (_BM, _OUT), 0) + i * _BM
    out_ref[...] = jnp.where((row < _N) & (d > 0), lax.rsqrt(d), 0.0)


def _dinv_call(deg2):
    return pl.pallas_call(
        _dinv_body,
        grid=(_NP // _BM,),
        in_specs=[pl.BlockSpec((_NC, _BM, _OUT), lambda i: (0, i, 0))],
        out_specs=pl.BlockSpec((_BM, _OUT), lambda i: (i, 0)),
        out_shape=jax.ShapeDtypeStruct((_NP, _OUT), jnp.float32),
    )(deg2)


def _mlp_body(x_ref, w1, b1, g1, be1, w2, b2, g2, be2, w3, b3,
              dinv_ref, t0, g0_out, hid_out):
    inv = np.float32(1.0 / np.sqrt(1.0 + _BN_EPS))
    h = jnp.dot(x_ref[...], w1[...], preferred_element_type=jnp.float32)
    h = jnp.maximum(h + b1[...], 0.0)
    h = h * (g1[...] * inv) + be1[...]
    h = jnp.dot(h, w2[...], preferred_element_type=jnp.float32)
    h = jnp.maximum(h + b2[...], 0.0)
    h = h * (g2[...] * inv) + be2[...]
    h = jnp.dot(h, w3[...], preferred_element_type=jnp.float32) + b3[...]
    hid_out[...] = t0[0, 0] * h
    g0_out[...] = dinv_ref[...] * h


def _mlp_call(xp, W1, b1, g1, be1, W2, b2, g2, be2, W3, b3, dinv, t0):
    full = lambda shape: pl.BlockSpec(shape, lambda i, s=shape: tuple(0 for _ in s))
    return pl.pallas_call(
        _mlp_body,
        grid=(_NP // _BM,),
        in_specs=[
            pl.BlockSpec((_BM, _IN_C), lambda i: (i, 0)),
            full((_IN_C, _HID)), full((1, _HID)), full((1, _HID)), full((1, _HID)),
            full((_HID, _HID)), full((1, _HID)), full((1, _HID)), full((1, _HID)),
            full((_HID, _OUT)), full((1, _OUT)),
            pl.BlockSpec((_BM, _OUT), lambda i: (i, 0)),
            full((1, 1)),
        ],
        out_specs=[pl.BlockSpec((_BM, _OUT), lambda i: (i, 0))] * 2,
        out_shape=[jax.ShapeDtypeStruct((_NP, _OUT), jnp.float32)] * 2,
    )(xp, W1, b1, g1, be1, W2, b2, g2, be2, W3, b3, dinv, t0)


def _comb_body(acc_ref, dinv_ref, hid_ref, tk, g_out, hid_out):
    h = dinv_ref[...] * (acc_ref[0] + acc_ref[1])
    hid_out[...] = hid_ref[...] + tk[0, 0] * h
    g_out[...] = dinv_ref[...] * h


def _comb_call(acc2, dinv, hidden, tk):
    return pl.pallas_call(
        _comb_body,
        grid=(_NP // _BM,),
        in_specs=[
            pl.BlockSpec((_NC, _BM, _OUT), lambda i: (0, i, 0)),
            pl.BlockSpec((_BM, _OUT), lambda i: (i, 0)),
            pl.BlockSpec((_BM, _OUT), lambda i: (i, 0)),
            pl.BlockSpec((1, 1), lambda i: (0, 0)),
        ],
        out_specs=[pl.BlockSpec((_BM, _OUT), lambda i: (i, 0))] * 2,
        out_shape=[jax.ShapeDtypeStruct((_NP, _OUT), jnp.float32)] * 2,
    )(acc2, dinv, hidden, tk)


def kernel(x, edge_index, W1, b1, g1, be1, W2, b2, g2, be2, W3, b3, temp):
    src = edge_index[0].astype(jnp.int32)
    dst = edge_index[1].astype(jnp.int32)
    loop = jnp.arange(_N, dtype=jnp.int32)
    npad = _EP - _N - src.shape[0]
    pad = _N + (jnp.arange(npad, dtype=jnp.int32) % (_NP - _N))
    srcp = jnp.concatenate([src, loop, pad]).reshape(_NC, _NS, _CPT, _CH)
    dstp = jnp.concatenate([dst, loop, pad]).reshape(_NC, _NS, _CPT, _CH)
    xp = jnp.pad(x, ((0, _NP - _N), (0, 0)))
    r = lambda v: v.reshape(1, -1)

    deg2 = _deg_kernel(dstp)
    dinv = _dinv_call(deg2)
    g, hidden = _mlp_call(xp, W1, r(b1), r(g1), r(be1), W2, r(b2), r(g2),
                          r(be2), W3, r(b3), dinv, temp[0:1].reshape(1, 1))
    for k in range(1, _K + 1):
        acc2 = _edge_kernel(g, srcp, dstp)
        g, hidden = _comb_call(acc2, dinv, hidden, temp[k:k + 1].reshape(1, 1))
    return hidden[:_N]


# speedup vs baseline: 11.2687x; 11.2687x over previous
import functools

import numpy as np
import jax
import jax.numpy as jnp
from jax import lax
from jax.experimental import pallas as pl
from jax.experimental.pallas import tpu as pltpu
from jax.experimental.pallas import tpu_sc as plsc

_N = 10000
_NP = 10240
_IN_C = 256
_HID = 256
_OUT = 128
_FW = 64
_K = 10
_BN_EPS = 1e-5

_NC, _NS = 2, 16
_CH = 128
_CPT = 81
_EP = _NC * _NS * _CPT * _CH
_ROWS_PT = _NP // _NS

_MESH = plsc.VectorSubcoreMesh(
    core_axis_name="c", subcore_axis_name="s", num_cores=_NC, num_subcores=_NS)
_MESH1 = plsc.VectorSubcoreMesh(
    core_axis_name="c", subcore_axis_name="s", num_cores=1, num_subcores=_NS)


def _fill_buf(buf, n_rows, width, value):
    vec = jnp.full((16,), value, jnp.float32)

    def body(i, _):
        for l in range(width // 16):
            buf[i, pl.ds(l * 16, 16)] = vec
        return 0

    lax.fori_loop(0, n_rows, body, 0)


@functools.partial(
    pl.kernel,
    out_type=jax.ShapeDtypeStruct((1, _NP, _OUT), jnp.float32),
    mesh=_MESH1,
    scratch_types=[
        pltpu.VMEM((2 * _CPT, _CH), jnp.int32),
        pltpu.VMEM((_CH, _OUT), jnp.float32),
        pltpu.VMEM_SHARED((_NP, _OUT), jnp.float32),
        pltpu.SemaphoreType.DMA,
    ],
)
def _deg_kernel(dst_hbm, out_hbm, dst_v, zo, acc_sh, sem):
    c = lax.axis_index("c")
    s = lax.axis_index("s")
    pltpu.sync_copy(dst_hbm.at[c, s], dst_v)
    _fill_buf(zo, _CH, _OUT, 0.0)
    for r in range(_ROWS_PT // _CH):
        pltpu.sync_copy(zo, acc_sh.at[pl.ds(s * _ROWS_PT + r * _CH, _CH)])
    _fill_buf(zo, _CH, _OUT, 1.0)
    plsc.subcore_barrier()

    def start(j):
        pltpu.async_copy(zo, acc_sh.at[dst_v.at[j]], sem, add=True)

    def drain(j):
        pltpu.make_async_copy(zo, acc_sh.at[dst_v.at[j]], sem).wait()

    def group(gi, _):
        for q in range(8):
            start(gi * 8 + q)
        for q in range(8):
            drain(gi * 8 + q)
        return 0

    n_ch = 2 * _CPT
    lax.fori_loop(0, n_ch // 8, group, 0)
    for j in range((n_ch // 8) * 8, n_ch):
        start(j)
        drain(j)
    plsc.subcore_barrier()
    pltpu.sync_copy(acc_sh.at[pl.ds(s * _ROWS_PT, _ROWS_PT)],
                    out_hbm.at[0, pl.ds(s * _ROWS_PT, _ROWS_PT)])


@functools.partial(
    pl.kernel,
    out_type=[jax.ShapeDtypeStruct((_NC, _NP, _FW), jnp.float32)] * 2,
    mesh=_MESH,
    compiler_params=pltpu.CompilerParams(use_tc_tiling_on_sc=False),
    scratch_types=[
        pltpu.VMEM((_CPT, _CH), jnp.int32),
        pltpu.VMEM((_CPT, _CH), jnp.int32),
        pltpu.VMEM((2, _CH, _FW), jnp.float32),
        pltpu.VMEM_SHARED((_NP, _FW), jnp.float32),
        pltpu.SemaphoreType.DMA((2,)),
        pltpu.SemaphoreType.DMA((2,)),
    ],
)
def _edge_kernel(g_a, g_b, src_hbm, dst_hbm, out_a, out_b,
                 src_v, dst_v, rows, acc_sh, gs, ss):
    c = lax.axis_index("c")
    s = lax.axis_index("s")
    pltpu.sync_copy(src_hbm.at[c, s], src_v)
    pltpu.sync_copy(dst_hbm.at[c, s], dst_v)

    for g_hbm, o_hbm in ((g_a, out_a), (g_b, out_b)):
        _fill_buf(rows.at[0], _CH, _FW, 0.0)
        for r in range(_ROWS_PT // _CH):
            pltpu.sync_copy(rows.at[0],
                            acc_sh.at[pl.ds(s * _ROWS_PT + r * _CH, _CH)])
        plsc.subcore_barrier()

        def g_start(j, b):
            pltpu.async_copy(g_hbm.at[src_v.at[j]], rows.at[b], gs.at[b])

        def g_wait(j, b):
            pltpu.make_async_copy(
                g_hbm.at[src_v.at[j]], rows.at[b], gs.at[b]).wait()

        def s_start(j, b):
            pltpu.async_copy(rows.at[b], acc_sh.at[dst_v.at[j]], ss.at[b],
                             add=True)

        def s_wait(j, b):
            pltpu.make_async_copy(
                rows.at[b], acc_sh.at[dst_v.at[j]], ss.at[b]).wait()

        def iteration(j, b):
            g_wait(j, b)
            s_start(j, b)
            ob = 1 - b
            if isinstance(j, int):
                if j >= 1:
                    s_wait(j - 1, ob)
                if j + 1 <= _CPT - 1:
                    g_start(j + 1, ob)
            else:
                s_wait(j - 1, ob)

                @pl.when(j + 1 <= _CPT - 1)
                def _():
                    g_start(j + 1, ob)

        g_start(0, 0)
        iteration(0, 0)

        def steady(m, _):
            iteration(2 * m + 1, 1)
            iteration(2 * m + 2, 0)
            return 0

        nsteady = (_CPT - 2) // 2
        lax.fori_loop(0, nsteady, steady, 0)
        for j in range(2 * nsteady + 1, _CPT):
            iteration(j, j % 2)
        s_wait(_CPT - 1, (_CPT - 1) % 2)
        plsc.subcore_barrier()
        pltpu.sync_copy(acc_sh.at[pl.ds(s * _ROWS_PT, _ROWS_PT)],
                        o_hbm.at[c, pl.ds(s * _ROWS_PT, _ROWS_PT)])


_BM = 1024


def _dinv_body(deg_ref, out_ref):
    i = pl.program_id(0)
    d = deg_ref[0]
    row = lax.broadcasted_iota(jnp.int32, (_BM, _OUT), 0) + i * _BM
    out_ref[...] = jnp.where((row < _N) & (d > 0), lax.rsqrt(d), 0.0)


def _dinv_call(deg2):
    return pl.pallas_call(
        _dinv_body,
        grid=(_NP // _BM,),
        in_specs=[pl.BlockSpec((1, _BM, _OUT), lambda i: (0, i, 0))],
        out_specs=pl.BlockSpec((_BM, _OUT), lambda i: (i, 0)),
        out_shape=jax.ShapeDtypeStruct((_NP, _OUT), jnp.float32),
    )(deg2)


def _mlp_body(x_ref, w1, b1, g1, be1, w2, b2, g2, be2, w3, b3,
              dinv_ref, t0, ga_out, gb_out, hid_out):
    inv = np.float32(1.0 / np.sqrt(1.0 + _BN_EPS))
    h = jnp.dot(x_ref[...], w1[...], preferred_element_type=jnp.float32)
    h = jnp.maximum(h + b1[...], 0.0)
    h = h * (g1[...] * inv) + be1[...]
    h = jnp.dot(h, w2[...], preferred_element_type=jnp.float32)
    h = jnp.maximum(h + b2[...], 0.0)
    h = h * (g2[...] * inv) + be2[...]
    h = jnp.dot(h, w3[...], preferred_element_type=jnp.float32) + b3[...]
    hid_out[...] = t0[0, 0] * h
    g = dinv_ref[...] * h
    ga_out[...] = g[:, :_FW]
    gb_out[...] = g[:, _FW:]


def _mlp_call(xp, W1, b1, g1, be1, W2, b2, g2, be2, W3, b3, dinv, t0):
    full = lambda shape: pl.BlockSpec(shape, lambda i, s=shape: tuple(0 for _ in s))
    return pl.pallas_call(
        _mlp_body,
        grid=(_NP // _BM,),
        in_specs=[
            pl.BlockSpec((_BM, _IN_C), lambda i: (i, 0)),
            full((_IN_C, _HID)), full((1, _HID)), full((1, _HID)), full((1, _HID)),
            full((_HID, _HID)), full((1, _HID)), full((1, _HID)), full((1, _HID)),
            full((_HID, _OUT)), full((1, _OUT)),
            pl.BlockSpec((_BM, _OUT), lambda i: (i, 0)),
            full((1, 1)),
        ],
        out_specs=[pl.BlockSpec((_BM, _FW), lambda i: (i, 0))] * 2
                  + [pl.BlockSpec((_BM, _OUT), lambda i: (i, 0))],
        out_shape=[jax.ShapeDtypeStruct((_NP, _FW), jnp.float32)] * 2
                  + [jax.ShapeDtypeStruct((_NP, _OUT), jnp.float32)],
    )(xp, W1, b1, g1, be1, W2, b2, g2, be2, W3, b3, dinv, t0)


def _comb_body(acc_a, acc_b, dinv_ref, hid_ref, tk, ga_out, gb_out, hid_out):
    da = dinv_ref[:, :_FW]
    db = dinv_ref[:, _FW:]
    ha = da * (acc_a[0] + acc_a[1])
    hb = db * (acc_b[0] + acc_b[1])
    hid_out[...] = hid_ref[...] + tk[0, 0] * jnp.concatenate([ha, hb], axis=1)
    ga_out[...] = da * ha
    gb_out[...] = db * hb


def _comb_call(acc_a, acc_b, dinv, hidden, tk):
    return pl.pallas_call(
        _comb_body,
        grid=(_NP // _BM,),
        in_specs=[
            pl.BlockSpec((_NC, _BM, _FW), lambda i: (0, i, 0)),
            pl.BlockSpec((_NC, _BM, _FW), lambda i: (0, i, 0)),
            pl.BlockSpec((_BM, _OUT), lambda i: (i, 0)),
            pl.BlockSpec((_BM, _OUT), lambda i: (i, 0)),
            pl.BlockSpec((1, 1), lambda i: (0, 0)),
        ],
        out_specs=[pl.BlockSpec((_BM, _FW), lambda i: (i, 0))] * 2
                  + [pl.BlockSpec((_BM, _OUT), lambda i: (i, 0))],
        out_shape=[jax.ShapeDtypeStruct((_NP, _FW), jnp.float32)] * 2
                  + [jax.ShapeDtypeStruct((_NP, _OUT), jnp.float32)],
    )(acc_a, acc_b, dinv, hidden, tk)


def kernel(x, edge_index, W1, b1, g1, be1, W2, b2, g2, be2, W3, b3, temp):
    src = edge_index[0].astype(jnp.int32)
    dst = edge_index[1].astype(jnp.int32)
    loop = jnp.arange(_N, dtype=jnp.int32)
    npad = _EP - _N - src.shape[0]
    pad = _N + (jnp.arange(npad, dtype=jnp.int32) % (_NP - _N))
    src_flat = jnp.concatenate([src, loop, pad])
    dst_flat = jnp.concatenate([dst, loop, pad])
    srcp = src_flat.reshape(_NC, _NS, _CPT, _CH)
    dstp = dst_flat.reshape(_NC, _NS, _CPT, _CH)
    dstp1 = dst_flat.reshape(1, _NS, 2 * _CPT, _CH)
    xp = jnp.pad(x, ((0, _NP - _N), (0, 0)))
    r = lambda v: v.reshape(1, -1)

    deg2 = _deg_kernel(dstp1)
    dinv = _dinv_call(deg2)
    g_a, g_b, hidden = _mlp_call(xp, W1, r(b1), r(g1), r(be1), W2, r(b2),
                                 r(g2), r(be2), W3, r(b3), dinv,
                                 temp[0:1].reshape(1, 1))
    for k in range(1, _K + 1):
        acc_a, acc_b = _edge_kernel(g_a, g_b, srcp, dstp)
        g_a, g_b, hidden = _comb_call(acc_a, acc_b, dinv, hidden,
                                      temp[k:k + 1].reshape(1, 1))
    return hidden[:_N]


# speedup vs baseline: 16.7172x; 1.4835x over previous
import functools

import numpy as np
import jax
import jax.numpy as jnp
from jax import lax
from jax.experimental import pallas as pl
from jax.experimental.pallas import tpu as pltpu
from jax.experimental.pallas import tpu_sc as plsc

_N = 10000
_NP = 10240
_IN_C = 256
_HID = 256
_OUT = 128
_FW = 64
_DW = 16
_K = 10
_BN_EPS = 1e-5

_NC, _NS = 2, 16
_CH = 128
_CPT = 162
_EP = _NS * _CPT * _CH
_ROWS_PT = _NP // _NS

_MESH = plsc.VectorSubcoreMesh(
    core_axis_name="c", subcore_axis_name="s", num_cores=_NC, num_subcores=_NS)
_MESH1 = plsc.VectorSubcoreMesh(
    core_axis_name="c", subcore_axis_name="s", num_cores=1, num_subcores=_NS)


def _fill_buf(buf, n_rows, width, value):
    vec = jnp.full((16,), value, jnp.float32)

    def body(i, _):
        for l in range(width // 16):
            buf[i, pl.ds(l * 16, 16)] = vec
        return 0

    lax.fori_loop(0, n_rows, body, 0)


@functools.partial(
    pl.kernel,
    out_type=jax.ShapeDtypeStruct((_NP, _DW), jnp.float32),
    mesh=_MESH1,
    compiler_params=pltpu.CompilerParams(use_tc_tiling_on_sc=False),
    scratch_types=[
        pltpu.VMEM((_CPT, _CH), jnp.int32),
        pltpu.VMEM((2, _CH, _DW), jnp.float32),
        pltpu.VMEM_SHARED((_NP, _DW), jnp.float32),
        pltpu.SemaphoreType.DMA,
    ],
)
def _deg_kernel(dst_hbm, out_hbm, dst_v, zo, acc_sh, sem):
    s = lax.axis_index("s")
    pltpu.sync_copy(dst_hbm.at[s], dst_v)
    _fill_buf(zo.at[0], _CH, _DW, 0.0)
    _fill_buf(zo.at[1], _CH, _DW, 1.0)
    for r in range(_ROWS_PT // _CH):
        pltpu.sync_copy(zo.at[0], acc_sh.at[pl.ds(s * _ROWS_PT + r * _CH, _CH)])
    plsc.subcore_barrier()

    def start(j):
        pltpu.async_copy(zo.at[1], acc_sh.at[dst_v.at[j]], sem, add=True)

    def drain(j):
        pltpu.make_async_copy(zo.at[1], acc_sh.at[dst_v.at[j]], sem).wait()

    def group(gi, _):
        for q in range(8):
            start(gi * 8 + q)
        for q in range(8):
            drain(gi * 8 + q)
        return 0

    lax.fori_loop(0, _CPT // 8, group, 0)
    for j in range((_CPT // 8) * 8, _CPT):
        start(j)
        drain(j)
    plsc.subcore_barrier()
    pltpu.sync_copy(acc_sh.at[pl.ds(s * _ROWS_PT, _ROWS_PT)],
                    out_hbm.at[pl.ds(s * _ROWS_PT, _ROWS_PT)])


@functools.partial(
    pl.kernel,
    out_type=[jax.ShapeDtypeStruct((_NP, _FW), jnp.float32)] * 2,
    mesh=_MESH,
    compiler_params=pltpu.CompilerParams(use_tc_tiling_on_sc=False),
    scratch_types=[
        pltpu.VMEM((_CPT, _CH), jnp.int32),
        pltpu.VMEM((_CPT, _CH), jnp.int32),
        pltpu.VMEM((4, _CH, _FW), jnp.float32),
        pltpu.VMEM_SHARED((_NP, _FW), jnp.float32),
        pltpu.SemaphoreType.DMA((4,)),
        pltpu.SemaphoreType.DMA((4,)),
    ],
)
def _edge_kernel(g_a, g_b, src_hbm, dst_hbm, out_a, out_b,
                 src_v, dst_v, rows, acc_sh, gs, ss):
    c = lax.axis_index("c")
    s = lax.axis_index("s")
    pltpu.sync_copy(src_hbm.at[s], src_v)
    pltpu.sync_copy(dst_hbm.at[s], dst_v)
    _fill_buf(rows.at[0], _CH, _FW, 0.0)
    for r in range(_ROWS_PT // _CH):
        pltpu.sync_copy(rows.at[0], acc_sh.at[pl.ds(s * _ROWS_PT + r * _CH, _CH)])
    plsc.subcore_barrier()

    def pipeline(g_hbm):
        def g_start(j, b):
            pltpu.async_copy(g_hbm.at[src_v.at[j]], rows.at[b], gs.at[b])

        def g_wait(j, b):
            pltpu.make_async_copy(
                g_hbm.at[src_v.at[j]], rows.at[b], gs.at[b]).wait()

        def s_start(j, b):
            pltpu.async_copy(rows.at[b], acc_sh.at[dst_v.at[j]], ss.at[b],
                             add=True)

        def s_wait(j, b):
            pltpu.make_async_copy(
                rows.at[b], acc_sh.at[dst_v.at[j]], ss.at[b]).wait()

        def iteration(j, b):
            g_wait(j, b)
            s_start(j, b)
            nb = (b + 2) % 4
            if isinstance(j, int):
                if j >= 2:
                    s_wait(j - 2, nb)
                if j + 2 <= _CPT - 1:
                    g_start(j + 2, nb)
            else:
                s_wait(j - 2, nb)

                @pl.when(j + 2 <= _CPT - 1)
                def _():
                    g_start(j + 2, nb)

        g_start(0, 0)
        g_start(1, 1)
        for j in range(4):
            iteration(j, j)

        def steady(m, _):
            for q in range(4):
                iteration(4 * m + 4 + q, q)
            return 0

        nsteady = (_CPT - 5) // 4
        lax.fori_loop(0, nsteady, steady, 0)
        for j in range(4 * nsteady + 4, _CPT):
            iteration(j, j % 4)
        s_wait(_CPT - 2, (_CPT - 2) % 4)
        s_wait(_CPT - 1, (_CPT - 1) % 4)

    @pl.when(c == 0)
    def _():
        pipeline(g_a)

    @pl.when(c == 1)
    def _():
        pipeline(g_b)

    plsc.subcore_barrier()
    sl = pl.ds(s * _ROWS_PT, _ROWS_PT)

    @pl.when(c == 0)
    def _():
        pltpu.sync_copy(acc_sh.at[sl], out_a.at[sl])

    @pl.when(c == 1)
    def _():
        pltpu.sync_copy(acc_sh.at[sl], out_b.at[sl])


_BM = 1024


def _dinv_body(deg_ref, out_ref):
    i = pl.program_id(0)
    d = deg_ref[...][:, :1]
    row = lax.broadcasted_iota(jnp.int32, (_BM, 1), 0) + i * _BM
    dv = jnp.where((row < _N) & (d > 0), lax.rsqrt(d), 0.0)
    out_ref[...] = jnp.broadcast_to(dv, (_BM, _OUT))


def _dinv_call(deg2):
    return pl.pallas_call(
        _dinv_body,
        grid=(_NP // _BM,),
        in_specs=[pl.BlockSpec((_BM, _DW), lambda i: (i, 0))],
        out_specs=pl.BlockSpec((_BM, _OUT), lambda i: (i, 0)),
        out_shape=jax.ShapeDtypeStruct((_NP, _OUT), jnp.float32),
    )(deg2)


def _mlp_body(x_ref, w1, b1, g1, be1, w2, b2, g2, be2, w3, b3,
              dinv_ref, t0, ga_out, gb_out, hid_out):
    inv = np.float32(1.0 / np.sqrt(1.0 + _BN_EPS))
    h = jnp.dot(x_ref[...], w1[...], preferred_element_type=jnp.float32)
    h = jnp.maximum(h + b1[...], 0.0)
    h = h * (g1[...] * inv) + be1[...]
    h = jnp.dot(h, w2[...], preferred_element_type=jnp.float32)
    h = jnp.maximum(h + b2[...], 0.0)
    h = h * (g2[...] * inv) + be2[...]
    h = jnp.dot(h, w3[...], preferred_element_type=jnp.float32) + b3[...]
    hid_out[...] = t0[0, 0] * h
    g = dinv_ref[...] * h
    ga_out[...] = g[:, :_FW]
    gb_out[...] = g[:, _FW:]


def _mlp_call(xp, W1, b1, g1, be1, W2, b2, g2, be2, W3, b3, dinv, t0):
    full = lambda shape: pl.BlockSpec(shape, lambda i, s=shape: tuple(0 for _ in s))
    return pl.pallas_call(
        _mlp_body,
        grid=(_NP // _BM,),
        in_specs=[
            pl.BlockSpec((_BM, _IN_C), lambda i: (i, 0)),
            full((_IN_C, _HID)), full((1, _HID)), full((1, _HID)), full((1, _HID)),
            full((_HID, _HID)), full((1, _HID)), full((1, _HID)), full((1, _HID)),
            full((_HID, _OUT)), full((1, _OUT)),
            pl.BlockSpec((_BM, _OUT), lambda i: (i, 0)),
            full((1, 1)),
        ],
        out_specs=[pl.BlockSpec((_BM, _FW), lambda i: (i, 0))] * 2
                  + [pl.BlockSpec((_BM, _OUT), lambda i: (i, 0))],
        out_shape=[jax.ShapeDtypeStruct((_NP, _FW), jnp.float32)] * 2
                  + [jax.ShapeDtypeStruct((_NP, _OUT), jnp.float32)],
    )(xp, W1, b1, g1, be1, W2, b2, g2, be2, W3, b3, dinv, t0)


def _comb_body(acc_a, acc_b, dinv_ref, hid_ref, tk, ga_out, gb_out, hid_out):
    da = dinv_ref[:, :_FW]
    db = dinv_ref[:, _FW:]
    ha = da * acc_a[...]
    hb = db * acc_b[...]
    hid_out[...] = hid_ref[...] + tk[0, 0] * jnp.concatenate([ha, hb], axis=1)
    ga_out[...] = da * ha
    gb_out[...] = db * hb


def _comb_call(acc_a, acc_b, dinv, hidden, tk):
    return pl.pallas_call(
        _comb_body,
        grid=(_NP // _BM,),
        in_specs=[
            pl.BlockSpec((_BM, _FW), lambda i: (i, 0)),
            pl.BlockSpec((_BM, _FW), lambda i: (i, 0)),
            pl.BlockSpec((_BM, _OUT), lambda i: (i, 0)),
            pl.BlockSpec((_BM, _OUT), lambda i: (i, 0)),
            pl.BlockSpec((1, 1), lambda i: (0, 0)),
        ],
        out_specs=[pl.BlockSpec((_BM, _FW), lambda i: (i, 0))] * 2
                  + [pl.BlockSpec((_BM, _OUT), lambda i: (i, 0))],
        out_shape=[jax.ShapeDtypeStruct((_NP, _FW), jnp.float32)] * 2
                  + [jax.ShapeDtypeStruct((_NP, _OUT), jnp.float32)],
    )(acc_a, acc_b, dinv, hidden, tk)


def kernel(x, edge_index, W1, b1, g1, be1, W2, b2, g2, be2, W3, b3, temp):
    src = edge_index[0].astype(jnp.int32)
    dst = edge_index[1].astype(jnp.int32)
    loop = jnp.arange(_N, dtype=jnp.int32)
    npad = _EP - _N - src.shape[0]
    pad = _N + (jnp.arange(npad, dtype=jnp.int32) % (_NP - _N))
    srcp = jnp.concatenate([src, loop, pad]).reshape(_NS, _CPT, _CH)
    dstp = jnp.concatenate([dst, loop, pad]).reshape(_NS, _CPT, _CH)
    xp = jnp.pad(x, ((0, _NP - _N), (0, 0)))
    r = lambda v: v.reshape(1, -1)

    deg = _deg_kernel(dstp)
    dinv = _dinv_call(deg)
    g_a, g_b, hidden = _mlp_call(xp, W1, r(b1), r(g1), r(be1), W2, r(b2),
                                 r(g2), r(be2), W3, r(b3), dinv,
                                 temp[0:1].reshape(1, 1))
    for k in range(1, _K + 1):
        acc_a, acc_b = _edge_kernel(g_a, g_b, srcp, dstp)
        g_a, g_b, hidden = _comb_call(acc_a, acc_b, dinv, hidden,
                                      temp[k:k + 1].reshape(1, 1))
    return hidden[:_N]


# speedup vs baseline: 19.2039x; 1.1487x over previous
import functools

import numpy as np
import jax
import jax.numpy as jnp
from jax import lax
from jax.experimental import pallas as pl
from jax.experimental.pallas import tpu as pltpu
from jax.experimental.pallas import tpu_sc as plsc

_N = 10000
_NP = 10240
_IN_C = 256
_HID = 256
_OUT = 128
_FW = 64
_DW = 16
_K = 10
_BN_EPS = 1e-5

_NC, _NS = 2, 16
_CH = 128
_CPT = 162
_EP = _NS * _CPT * _CH
_ROWS_PT = _NP // _NS

_MESH = plsc.VectorSubcoreMesh(
    core_axis_name="c", subcore_axis_name="s", num_cores=_NC, num_subcores=_NS)
_MESH1 = plsc.VectorSubcoreMesh(
    core_axis_name="c", subcore_axis_name="s", num_cores=1, num_subcores=_NS)


def _fill_buf(buf, n_rows, width, value):
    vec = jnp.full((16,), value, jnp.float32)

    def body(i, _):
        for l in range(width // 16):
            buf[i, pl.ds(l * 16, 16)] = vec
        return 0

    lax.fori_loop(0, n_rows, body, 0)


@functools.partial(
    pl.kernel,
    out_type=jax.ShapeDtypeStruct((_NP, _DW), jnp.float32),
    mesh=_MESH1,
    compiler_params=pltpu.CompilerParams(use_tc_tiling_on_sc=False),
    scratch_types=[
        pltpu.VMEM((_CPT, _CH), jnp.int32),
        pltpu.VMEM((2, _CH, _DW), jnp.float32),
        pltpu.VMEM_SHARED((_NP, _DW), jnp.float32),
        pltpu.SemaphoreType.DMA,
    ],
)
def _deg_kernel(dst_hbm, out_hbm, dst_v, zo, acc_sh, sem):
    s = lax.axis_index("s")
    pltpu.sync_copy(dst_hbm.at[s], dst_v)
    _fill_buf(zo.at[0], _CH, _DW, 0.0)
    _fill_buf(zo.at[1], _CH, _DW, 1.0)
    for r in range(_ROWS_PT // _CH):
        pltpu.sync_copy(zo.at[0], acc_sh.at[pl.ds(s * _ROWS_PT + r * _CH, _CH)])
    plsc.subcore_barrier()

    def start(j):
        pltpu.async_copy(zo.at[1], acc_sh.at[dst_v.at[j]], sem, add=True)

    def drain(j):
        pltpu.make_async_copy(zo.at[1], acc_sh.at[dst_v.at[j]], sem).wait()

    def group(gi, _):
        for q in range(8):
            start(gi * 8 + q)
        for q in range(8):
            drain(gi * 8 + q)
        return 0

    lax.fori_loop(0, _CPT // 8, group, 0)
    for j in range((_CPT // 8) * 8, _CPT):
        start(j)
        drain(j)
    plsc.subcore_barrier()
    pltpu.sync_copy(acc_sh.at[pl.ds(s * _ROWS_PT, _ROWS_PT)],
                    out_hbm.at[pl.ds(s * _ROWS_PT, _ROWS_PT)])


@functools.partial(
    pl.kernel,
    out_type=[jax.ShapeDtypeStruct((_NP, _FW), jnp.float32)] * 2,
    mesh=_MESH,
    compiler_params=pltpu.CompilerParams(use_tc_tiling_on_sc=False),
    scratch_types=[
        pltpu.VMEM((_CPT, _CH), jnp.int32),
        pltpu.VMEM((_CPT, _CH), jnp.int32),
        pltpu.VMEM((5, _CH, _FW), jnp.float32),
        pltpu.VMEM_SHARED((_NP, _FW), jnp.float32),
        pltpu.SemaphoreType.DMA((5,)),
        pltpu.SemaphoreType.DMA((5,)),
    ],
)
def _edge_kernel(g_a, g_b, src_hbm, dst_hbm, out_a, out_b,
                 src_v, dst_v, rows, acc_sh, gs, ss):
    c = lax.axis_index("c")
    s = lax.axis_index("s")
    pltpu.sync_copy(src_hbm.at[s], src_v)
    pltpu.sync_copy(dst_hbm.at[s], dst_v)
    _fill_buf(rows.at[0], _CH, _FW, 0.0)
    for r in range(_ROWS_PT // _CH):
        pltpu.sync_copy(rows.at[0], acc_sh.at[pl.ds(s * _ROWS_PT + r * _CH, _CH)])
    plsc.subcore_barrier()

    def pipeline(g_hbm):
        def g_start(j, b):
            pltpu.async_copy(g_hbm.at[src_v.at[j]], rows.at[b], gs.at[b])

        def g_wait(j, b):
            pltpu.make_async_copy(
                g_hbm.at[src_v.at[j]], rows.at[b], gs.at[b]).wait()

        def s_start(j, b):
            pltpu.async_copy(rows.at[b], acc_sh.at[dst_v.at[j]], ss.at[b],
                             add=True)

        def s_wait(j, b):
            pltpu.make_async_copy(
                rows.at[b], acc_sh.at[dst_v.at[j]], ss.at[b]).wait()

        def iteration(j, b):
            g_wait(j, b)
            s_start(j, b)
            nb = (b + 3) % 5
            if isinstance(j, int):
                if j >= 2:
                    s_wait(j - 2, nb)
                if j + 3 <= _CPT - 1:
                    g_start(j + 3, nb)
            else:
                s_wait(j - 2, nb)

                @pl.when(j + 3 <= _CPT - 1)
                def _():
                    g_start(j + 3, nb)

        g_start(0, 0)
        g_start(1, 1)
        g_start(2, 2)
        for j in range(5):
            iteration(j, j)

        def steady(m, _):
            for q in range(5):
                iteration(5 * m + 5 + q, q)
            return 0

        nsteady = (_CPT - 6) // 5
        lax.fori_loop(0, nsteady, steady, 0)
        for j in range(5 * nsteady + 5, _CPT):
            iteration(j, j % 5)
        s_wait(_CPT - 2, (_CPT - 2) % 5)
        s_wait(_CPT - 1, (_CPT - 1) % 5)

    @pl.when(c == 0)
    def _():
        pipeline(g_a)

    @pl.when(c == 1)
    def _():
        pipeline(g_b)

    plsc.subcore_barrier()
    sl = pl.ds(s * _ROWS_PT, _ROWS_PT)

    @pl.when(c == 0)
    def _():
        pltpu.sync_copy(acc_sh.at[sl], out_a.at[sl])

    @pl.when(c == 1)
    def _():
        pltpu.sync_copy(acc_sh.at[sl], out_b.at[sl])


_BM = 1024


def _dinv_body(deg_ref, out_ref):
    i = pl.program_id(0)
    d = deg_ref[...][:, :1]
    row = lax.broadcasted_iota(jnp.int32, (_BM, 1), 0) + i * _BM
    dv = jnp.where((row < _N) & (d > 0), lax.rsqrt(d), 0.0)
    out_ref[...] = jnp.broadcast_to(dv, (_BM, _OUT))


def _dinv_call(deg2):
    return pl.pallas_call(
        _dinv_body,
        grid=(_NP // _BM,),
        in_specs=[pl.BlockSpec((_BM, _DW), lambda i: (i, 0))],
        out_specs=pl.BlockSpec((_BM, _OUT), lambda i: (i, 0)),
        out_shape=jax.ShapeDtypeStruct((_NP, _OUT), jnp.float32),
    )(deg2)


def _mlp_body(x_ref, w1, b1, g1, be1, w2, b2, g2, be2, w3, b3,
              dinv_ref, t0, ga_out, gb_out, hid_out):
    inv = np.float32(1.0 / np.sqrt(1.0 + _BN_EPS))
    h = jnp.dot(x_ref[...], w1[...], preferred_element_type=jnp.float32)
    h = jnp.maximum(h + b1[...], 0.0)
    h = h * (g1[...] * inv) + be1[...]
    h = jnp.dot(h, w2[...], preferred_element_type=jnp.float32)
    h = jnp.maximum(h + b2[...], 0.0)
    h = h * (g2[...] * inv) + be2[...]
    h = jnp.dot(h, w3[...], preferred_element_type=jnp.float32) + b3[...]
    hid_out[...] = t0[0, 0] * h
    g = dinv_ref[...] * h
    ga_out[...] = g[:, :_FW]
    gb_out[...] = g[:, _FW:]


def _mlp_call(xp, W1, b1, g1, be1, W2, b2, g2, be2, W3, b3, dinv, t0):
    full = lambda shape: pl.BlockSpec(shape, lambda i, s=shape: tuple(0 for _ in s))
    return pl.pallas_call(
        _mlp_body,
        grid=(_NP // _BM,),
        in_specs=[
            pl.BlockSpec((_BM, _IN_C), lambda i: (i, 0)),
            full((_IN_C, _HID)), full((1, _HID)), full((1, _HID)), full((1, _HID)),
            full((_HID, _HID)), full((1, _HID)), full((1, _HID)), full((1, _HID)),
            full((_HID, _OUT)), full((1, _OUT)),
            pl.BlockSpec((_BM, _OUT), lambda i: (i, 0)),
            full((1, 1)),
        ],
        out_specs=[pl.BlockSpec((_BM, _FW), lambda i: (i, 0))] * 2
                  + [pl.BlockSpec((_BM, _OUT), lambda i: (i, 0))],
        out_shape=[jax.ShapeDtypeStruct((_NP, _FW), jnp.float32)] * 2
                  + [jax.ShapeDtypeStruct((_NP, _OUT), jnp.float32)],
    )(xp, W1, b1, g1, be1, W2, b2, g2, be2, W3, b3, dinv, t0)


def _comb_body(acc_a, acc_b, dinv_ref, hid_ref, tk, ga_out, gb_out, hid_out):
    da = dinv_ref[:, :_FW]
    db = dinv_ref[:, _FW:]
    ha = da * acc_a[...]
    hb = db * acc_b[...]
    hid_out[...] = hid_ref[...] + tk[0, 0] * jnp.concatenate([ha, hb], axis=1)
    ga_out[...] = da * ha
    gb_out[...] = db * hb


def _comb_call(acc_a, acc_b, dinv, hidden, tk):
    return pl.pallas_call(
        _comb_body,
        grid=(_NP // _BM,),
        in_specs=[
            pl.BlockSpec((_BM, _FW), lambda i: (i, 0)),
            pl.BlockSpec((_BM, _FW), lambda i: (i, 0)),
            pl.BlockSpec((_BM, _OUT), lambda i: (i, 0)),
            pl.BlockSpec((_BM, _OUT), lambda i: (i, 0)),
            pl.BlockSpec((1, 1), lambda i: (0, 0)),
        ],
        out_specs=[pl.BlockSpec((_BM, _FW), lambda i: (i, 0))] * 2
                  + [pl.BlockSpec((_BM, _OUT), lambda i: (i, 0))],
        out_shape=[jax.ShapeDtypeStruct((_NP, _FW), jnp.float32)] * 2
                  + [jax.ShapeDtypeStruct((_NP, _OUT), jnp.float32)],
    )(acc_a, acc_b, dinv, hidden, tk)


def kernel(x, edge_index, W1, b1, g1, be1, W2, b2, g2, be2, W3, b3, temp):
    src = edge_index[0].astype(jnp.int32)
    dst = edge_index[1].astype(jnp.int32)
    loop = jnp.arange(_N, dtype=jnp.int32)
    npad = _EP - _N - src.shape[0]
    pad = _N + (jnp.arange(npad, dtype=jnp.int32) % (_NP - _N))
    srcp = jnp.concatenate([src, loop, pad]).reshape(_NS, _CPT, _CH)
    dstp = jnp.concatenate([dst, loop, pad]).reshape(_NS, _CPT, _CH)
    xp = jnp.pad(x, ((0, _NP - _N), (0, 0)))
    r = lambda v: v.reshape(1, -1)

    deg = _deg_kernel(dstp)
    dinv = _dinv_call(deg)
    g_a, g_b, hidden = _mlp_call(xp, W1, r(b1), r(g1), r(be1), W2, r(b2),
                                 r(g2), r(be2), W3, r(b3), dinv,
                                 temp[0:1].reshape(1, 1))
    for k in range(1, _K + 1):
        acc_a, acc_b = _edge_kernel(g_a, g_b, srcp, dstp)
        g_a, g_b, hidden = _comb_call(acc_a, acc_b, dinv, hidden,
                                      temp[k:k + 1].reshape(1, 1))
    return hidden[:_N]
